# Initial kernel scaffold; baseline (speedup 1.0000x reference)
import jax
import jax.numpy as jnp
from jax.experimental import pallas as pl


def kernel(obs, edge_index, edge_attr, params):
    raise NotImplementedError("write your pallas kernel here")


# speedup vs baseline: 2.4794x; 2.4794x over previous
import functools

import jax
import jax.numpy as jnp
from jax import lax
from jax.experimental import pallas as pl
from jax.experimental.pallas import tpu as pltpu
from jax.experimental.pallas import tpu_sc as plsc

_N = 50000
_E = 800000
_EMB = 64
_HID = 128
_EDGE = 8

_NC = 2
_NS = 16
_NW = _NC * _NS

_EP = 819200
_GCHUNK = 1024
_SCHUNK = 512
_HALF = _N // 2
_ACC_ROWS = 25600

_BLK_N = 2000
_BLK_E = 2048


def _sc_mesh():
    return plsc.VectorSubcoreMesh(
        core_axis_name="c", subcore_axis_name="s",
        num_cores=_NC, num_subcores=_NS)


def _gather(h, src_g, dst_g):
    n_chunks = _EP // _NW // _GCHUNK

    @functools.partial(
        pl.kernel,
        out_type=(jax.ShapeDtypeStruct((_EP, _EMB), jnp.float32),
                  jax.ShapeDtypeStruct((_EP, _EMB), jnp.float32)),
        mesh=_sc_mesh(),
        scratch_types=[
            pltpu.VMEM((_GCHUNK,), jnp.int32),
            pltpu.VMEM((_GCHUNK, _EMB), jnp.float32),
            pltpu.SemaphoreType.DMA,
        ],
        compiler_params=pltpu.CompilerParams(use_tc_tiling_on_sc=False),
    )
    def k(h_hbm, src_hbm, dst_hbm, xj_hbm, xi_hbm, idx_v, rows_v, sem):
        c = lax.axis_index("c")
        s = lax.axis_index("s")
        wid = s * _NC + c
        base = wid * (_EP // _NW)

        def one(idx_hbm, out_hbm, off):
            pltpu.sync_copy(idx_hbm.at[pl.ds(off, _GCHUNK)], idx_v)
            cps = [
                pltpu.async_copy(
                    h_hbm.at[idx_v.at[pl.ds(j * 128, 128)]],
                    rows_v.at[pl.ds(j * 128, 128)], sem)
                for j in range(_GCHUNK // 128)
            ]
            for cp in cps:
                cp.wait()
            pltpu.sync_copy(rows_v, out_hbm.at[pl.ds(off, _GCHUNK)])

        def step(i, carry):
            off = base + i * _GCHUNK
            one(src_hbm, xj_hbm, off)
            one(dst_hbm, xi_hbm, off)
            return carry

        lax.fori_loop(0, n_chunks, step, 0)

    return k(h, src_g, dst_g)


def _scatter(wmsg, dst_s):
    n_chunks = _EP // _NS // _SCHUNK
    fw = _EMB // 2

    @functools.partial(
        pl.kernel,
        out_type=jax.ShapeDtypeStruct((_N, _EMB), jnp.float32),
        mesh=_sc_mesh(),
        scratch_types=[
            pltpu.VMEM((_SCHUNK,), jnp.int32),
            pltpu.VMEM((_SCHUNK // 128, 128), jnp.int32),
            pltpu.VMEM((_SCHUNK, fw), jnp.float32),
            pltpu.VMEM_SHARED((_ACC_ROWS, fw), jnp.float32),
        ],
        compiler_params=pltpu.CompilerParams(use_tc_tiling_on_sc=False),
    )
    def k(w_hbm, d_hbm, out_hbm, raw_v, idx2_v, vals_v, acc):
        c = lax.axis_index("c")
        s = lax.axis_index("s")
        lane = lax.iota(jnp.int32, 16)
        base = s * (_EP // _NS)
        nodes0 = c * _HALF
        zb = s * (_ACC_ROWS // _NS)

        for f in range(2):
            def zrow(r, carry):
                for t in range(fw // 16):
                    vals_v[r, pl.ds(t * 16, 16)] = jnp.zeros(
                        (16,), jnp.float32)
                return carry
            lax.fori_loop(0, _SCHUNK, zrow, 0)
            for t in range(3):
                pltpu.sync_copy(vals_v,
                                acc.at[pl.ds(zb + t * _SCHUNK, _SCHUNK)])
            pltpu.sync_copy(vals_v.at[pl.ds(0, 64)],
                            acc.at[pl.ds(zb + 3 * _SCHUNK, 64)])
            plsc.subcore_barrier()

            def step(i, carry):
                off = base + i * _SCHUNK
                pltpu.sync_copy(d_hbm.at[pl.ds(off, _SCHUNK)], raw_v)
                pltpu.sync_copy(
                    w_hbm.at[pl.ds(off, _SCHUNK), pl.ds(f * fw, fw)], vals_v)
                for kv in range(_SCHUNK // 16):
                    v = raw_v[pl.ds(kv * 16, 16)]
                    local = v - nodes0
                    inr = (local >= 0) & (local < _HALF)
                    pad = _HALF + s * 32 + ((lane + kv) & 31)
                    idx2_v[kv // 8, pl.ds((kv % 8) * 16, 16)] = (
                        jnp.where(inr, local, pad))
                for j in range(_SCHUNK // 128):
                    pltpu.sync_copy(vals_v.at[pl.ds(j * 128, 128)],
                                    acc.at[idx2_v.at[j]], add=True)
                return carry

            lax.fori_loop(0, n_chunks, step, 0)
            plsc.subcore_barrier()

            @pl.when(s < 8)
            def _():
                rows = _HALF // 8
                pltpu.sync_copy(
                    acc.at[pl.ds(s * rows, rows)],
                    out_hbm.at[pl.ds(nodes0 + s * rows, rows),
                               pl.ds(f * fw, fw)])
            plsc.subcore_barrier()

    return k(wmsg, dst_s)


def _row_spec(blk, d):
    return pl.BlockSpec((blk, d), lambda i: (i, 0))


def _full_spec(d1, d2):
    return pl.BlockSpec((d1, d2), lambda i: (0, 0))


def _ln(x, g, b):
    m = jnp.mean(x, axis=-1, keepdims=True)
    v = jnp.mean((x - m) ** 2, axis=-1, keepdims=True)
    return (x - m) * lax.rsqrt(v + 1e-5) * g + b


def _enc_body(o_ref, w1, b1, g1, be1, w2, b2, out_ref):
    h = jnp.dot(o_ref[...], w1[...], preferred_element_type=jnp.float32)
    h = h + b1[...]
    h = jnp.maximum(_ln(h, g1[...], be1[...]), 0.0)
    h2 = jnp.dot(h, w2[...], preferred_element_type=jnp.float32) + b2[...]
    out_ref[...] = jnp.maximum(h2, 0.0)


def _encoder(obs10, p):
    return pl.pallas_call(
        _enc_body,
        grid=(_N // _BLK_N,),
        in_specs=[
            _row_spec(_BLK_N, 10),
            _full_spec(10, _HID), _full_spec(1, _HID),
            _full_spec(1, _HID), _full_spec(1, _HID),
            _full_spec(_HID, _EMB), _full_spec(1, _EMB),
        ],
        out_specs=_row_spec(_BLK_N, _EMB),
        out_shape=jax.ShapeDtypeStruct((_N, _EMB), jnp.float32),
    )(obs10, p['enc_w1'], p['enc_b1'].reshape(1, -1),
      p['enc_g1'].reshape(1, -1), p['enc_be1'].reshape(1, -1),
      p['enc_w2'], p['enc_b2'].reshape(1, -1))


def _edge_body(xi, xj, ea, wi, wj, we, b1, w2, b2, out):
    pre = jnp.dot(xi[...], wi[...], preferred_element_type=jnp.float32)
    pre = pre + jnp.dot(xj[...], wj[...], preferred_element_type=jnp.float32)
    pre = pre + jnp.dot(ea[...], we[...], preferred_element_type=jnp.float32)
    h1 = jnp.maximum(pre + b1[...], 0.0)
    msg = jnp.dot(h1, w2[...], preferred_element_type=jnp.float32) + b2[...]
    out[...] = 0.25 * msg


def _edge_mlp(xi, xj, ea, wi, wj, we, b1, w2, b2):
    return pl.pallas_call(
        _edge_body,
        grid=(_EP // _BLK_E,),
        in_specs=[
            _row_spec(_BLK_E, _EMB), _row_spec(_BLK_E, _EMB),
            _row_spec(_BLK_E, _EDGE),
            _full_spec(_EMB, _HID), _full_spec(_EMB, _HID),
            _full_spec(_EDGE, _HID), _full_spec(1, _HID),
            _full_spec(_HID, _EMB), _full_spec(1, _EMB),
        ],
        out_specs=_row_spec(_BLK_E, _EMB),
        out_shape=jax.ShapeDtypeStruct((_EP, _EMB), jnp.float32),
    )(xi, xj, ea, wi, wj, we, b1.reshape(1, -1), w2, b2.reshape(1, -1))


def _upd_body(h, a, w1h, w1a, b1, w2, b2, g, b, out):
    u = jnp.dot(h[...], w1h[...], preferred_element_type=jnp.float32)
    u = u + jnp.dot(a[...], w1a[...], preferred_element_type=jnp.float32)
    u = jnp.maximum(u + b1[...], 0.0)
    upd = jnp.dot(u, w2[...], preferred_element_type=jnp.float32) + b2[...]
    out[...] = _ln(h[...] + upd, g[...], b[...])


def _update(h, aggr, w1h, w1a, b1, w2, b2, g, b):
    return pl.pallas_call(
        _upd_body,
        grid=(_N // _BLK_N,),
        in_specs=[
            _row_spec(_BLK_N, _EMB), _row_spec(_BLK_N, _EMB),
            _full_spec(_EMB, _HID), _full_spec(_EMB, _HID),
            _full_spec(1, _HID),
            _full_spec(_HID, _EMB), _full_spec(1, _EMB),
            _full_spec(1, _EMB), _full_spec(1, _EMB),
        ],
        out_specs=_row_spec(_BLK_N, _EMB),
        out_shape=jax.ShapeDtypeStruct((_N, _EMB), jnp.float32),
    )(h, aggr, w1h, w1a, b1.reshape(1, -1), w2, b2.reshape(1, -1),
      g.reshape(1, -1), b.reshape(1, -1))


def _out_body(h, w1, b1, w2, b2, out):
    u = jnp.dot(h[...], w1[...], preferred_element_type=jnp.float32)
    u = jnp.maximum(u + b1[...], 0.0)
    out[...] = jnp.dot(u, w2[...], preferred_element_type=jnp.float32) + b2[...]


def _output(h, w1, b1, w2, b2):
    return pl.pallas_call(
        _out_body,
        grid=(_N // _BLK_N,),
        in_specs=[
            _row_spec(_BLK_N, _EMB),
            _full_spec(_EMB, _HID), _full_spec(1, _HID),
            _full_spec(_HID, _EMB), _full_spec(1, _EMB),
        ],
        out_specs=_row_spec(_BLK_N, _EMB),
        out_shape=jax.ShapeDtypeStruct((_N, _EMB), jnp.float32),
    )(h, w1, b1.reshape(1, -1), w2, b2.reshape(1, -1))


def kernel(obs, edge_index, edge_attr, params):
    p = params
    src = edge_index[0]
    dst = edge_index[1]
    padn = _EP - _E
    ar = jnp.arange(padn, dtype=jnp.int32)
    src_g = jnp.concatenate([src, ar % _N])
    dst_g = jnp.concatenate([dst, (ar * 7 + 13) % _N])
    dst_s = jnp.concatenate([dst, _N + (ar & 255)])
    ea_p = jnp.concatenate(
        [edge_attr, jnp.zeros((padn, _EDGE), jnp.float32)], axis=0)

    h = _encoder(obs[:, :10], p)
    for lp in p['layers']:
        xj, xi = _gather(h, src_g, dst_g)
        w1 = lp['msg_w1']
        wmsg = _edge_mlp(xi, xj, ea_p,
                         w1[:_EMB], w1[_EMB:2 * _EMB], w1[2 * _EMB:],
                         lp['msg_b1'], lp['msg_w2'], lp['msg_b2'])
        aggr = _scatter(wmsg, dst_s)
        uw1 = lp['upd_w1']
        h = _update(h, aggr, uw1[:_EMB], uw1[_EMB:], lp['upd_b1'],
                    lp['upd_w2'], lp['upd_b2'], lp['ln_g'], lp['ln_b'])
    return _output(h, p['out_w1'], p['out_b1'], p['out_w2'], p['out_b2'])


# speedup vs baseline: 4.5197x; 1.8229x over previous
import functools

import jax
import jax.numpy as jnp
from jax import lax
from jax.experimental import pallas as pl
from jax.experimental.pallas import tpu as pltpu
from jax.experimental.pallas import tpu_sc as plsc

_N = 50000
_E = 800000
_EMB = 64
_HID = 128
_EDGE = 8

_NC = 2
_NS = 16
_NW = _NC * _NS

_GCHUNK = 1024
_G_PER_W = _E // _NW
_G_FULL = _G_PER_W // _GCHUNK
_G_TAIL = _G_PER_W - _G_FULL * _GCHUNK

_SCHUNK = 512
_S_PER_W = _E // _NS
_S_FULL = _S_PER_W // _SCHUNK
_S_TAIL = _S_PER_W - _S_FULL * _SCHUNK

_HALF = _N // 2
_QUART = _HALF // 2
_ODD_BASE = 12800
_PAD_BASE = 25300
_ACC_ROWS = 25600

_BLK_N = 1000
_BLK_E = 3200


def _sc_mesh():
    return plsc.VectorSubcoreMesh(
        core_axis_name="c", subcore_axis_name="s",
        num_cores=_NC, num_subcores=_NS)


def _gather(h64, src, dst):
    @functools.partial(
        pl.kernel,
        out_type=jax.ShapeDtypeStruct((_E, 2 * _EMB), jnp.float32),
        mesh=_sc_mesh(),
        scratch_types=[
            pltpu.VMEM((_GCHUNK,), jnp.int32),
            pltpu.VMEM((_GCHUNK, _EMB), jnp.float32),
            pltpu.SemaphoreType.DMA,
        ],
        compiler_params=pltpu.CompilerParams(use_tc_tiling_on_sc=False),
    )
    def k(h_hbm, src_hbm, dst_hbm, cat_hbm, idx_v, rows_v, sem):
        c = lax.axis_index("c")
        s = lax.axis_index("s")
        wid = s * _NC + c
        base = wid * _G_PER_W

        def one(idx_hbm, col, off, n, nidx):
            pltpu.sync_copy(idx_hbm.at[pl.ds(off, n)], idx_v.at[pl.ds(0, n)])
            cps = [pltpu.async_copy(
                h_hbm.at[idx_v.at[pl.ds(st, ln)]],
                rows_v.at[pl.ds(st, ln)], sem) for (st, ln) in nidx]
            for cp in cps:
                cp.wait()
            pltpu.sync_copy(rows_v.at[pl.ds(0, n)],
                            cat_hbm.at[pl.ds(off, n), pl.ds(col, _EMB)])

        full_slices = [(j * 128, 128) for j in range(_GCHUNK // 128)]
        tail_slices = [(j * 128, 128) for j in range(_G_TAIL // 128)]
        if _G_TAIL % 128:
            tail_slices.append((_G_TAIL - _G_TAIL % 128, _G_TAIL % 128))

        def step(i, carry):
            off = base + i * _GCHUNK
            one(dst_hbm, 0, off, _GCHUNK, full_slices)
            one(src_hbm, _EMB, off, _GCHUNK, full_slices)
            return carry

        lax.fori_loop(0, _G_FULL, step, 0)
        toff = base + _G_FULL * _GCHUNK
        one(dst_hbm, 0, toff, _G_TAIL, tail_slices)
        one(src_hbm, _EMB, toff, _G_TAIL, tail_slices)

    return k(h64, src, dst)


def _scatter(wmsg128, dst):
    fw = _EMB // 2

    @functools.partial(
        pl.kernel,
        out_type=jax.ShapeDtypeStruct((_HALF, 2 * _EMB), jnp.float32),
        mesh=_sc_mesh(),
        scratch_types=[
            pltpu.VMEM((_SCHUNK,), jnp.int32),
            pltpu.VMEM((_SCHUNK // 128, 128), jnp.int32),
            pltpu.VMEM((_SCHUNK, fw), jnp.float32),
            pltpu.VMEM_SHARED((_ACC_ROWS, fw), jnp.float32),
        ],
        compiler_params=pltpu.CompilerParams(use_tc_tiling_on_sc=False),
    )
    def k(w_hbm, d_hbm, out_hbm, raw_v, idx2_v, vals_v, acc):
        c = lax.axis_index("c")
        s = lax.axis_index("s")
        lane = lax.iota(jnp.int32, 16)
        base = s * _S_PER_W
        nodes0 = c * _HALF
        zb = s * (_ACC_ROWS // _NS)

        def fixup(kv, v):
            local = v - nodes0
            inr = (local >= 0) & (local < _HALF)
            lrow = (local >> 1) + (local & 1) * _ODD_BASE
            pad = _PAD_BASE + s * 16 + ((lane + kv) & 15)
            idx2_v[kv // 8, pl.ds((kv % 8) * 16, 16)] = (
                jnp.where(inr, lrow, pad))

        for f in range(2):
            def zrow(r, carry):
                for t in range(fw // 16):
                    vals_v[r, pl.ds(t * 16, 16)] = jnp.zeros(
                        (16,), jnp.float32)
                return carry
            lax.fori_loop(0, _SCHUNK, zrow, 0)
            for t in range(3):
                pltpu.sync_copy(vals_v,
                                acc.at[pl.ds(zb + t * _SCHUNK, _SCHUNK)])
            pltpu.sync_copy(vals_v.at[pl.ds(0, 64)],
                            acc.at[pl.ds(zb + 3 * _SCHUNK, 64)])
            plsc.subcore_barrier()

            def step(i, carry):
                off = base + i * _SCHUNK
                pltpu.sync_copy(d_hbm.at[pl.ds(off, _SCHUNK)], raw_v)
                pltpu.sync_copy(
                    w_hbm.at[pl.ds(off, _SCHUNK), pl.ds(f * fw, fw)], vals_v)
                for kv in range(_SCHUNK // 16):
                    fixup(kv, raw_v[pl.ds(kv * 16, 16)])
                for j in range(_SCHUNK // 128):
                    pltpu.sync_copy(vals_v.at[pl.ds(j * 128, 128)],
                                    acc.at[idx2_v.at[j]], add=True)
                return carry

            lax.fori_loop(0, _S_FULL, step, 0)
            toff = base + _S_FULL * _SCHUNK
            pltpu.sync_copy(d_hbm.at[pl.ds(toff, _S_TAIL)],
                            raw_v.at[pl.ds(0, _S_TAIL)])
            pltpu.sync_copy(
                w_hbm.at[pl.ds(toff, _S_TAIL), pl.ds(f * fw, fw)],
                vals_v.at[pl.ds(0, _S_TAIL)])
            for kv in range(_S_TAIL // 16):
                fixup(kv, raw_v[pl.ds(kv * 16, 16)])
            for kv in range(_S_TAIL // 16, _SCHUNK // 16):
                pad = _PAD_BASE + s * 16 + ((lane + kv) & 15)
                idx2_v[kv // 8, pl.ds((kv % 8) * 16, 16)] = pad
            for j in range(_SCHUNK // 128):
                pltpu.sync_copy(vals_v.at[pl.ds(j * 128, 128)],
                                acc.at[idx2_v.at[j]], add=True)
            plsc.subcore_barrier()

            rows = _QUART // 4
            @pl.when(s < 4)
            def _():
                pltpu.sync_copy(
                    acc.at[pl.ds(s * rows, rows)],
                    out_hbm.at[pl.ds(c * _QUART + s * rows, rows),
                               pl.ds(f * fw, fw)])

            @pl.when((s >= 4) & (s < 8))
            def _():
                pltpu.sync_copy(
                    acc.at[pl.ds(_ODD_BASE + (s - 4) * rows, rows)],
                    out_hbm.at[pl.ds(c * _QUART + (s - 4) * rows, rows),
                               pl.ds(_EMB + f * fw, fw)])
            plsc.subcore_barrier()

    return k(wmsg128, dst)


def _full_spec(d1, d2):
    return pl.BlockSpec((d1, d2), lambda i: (0, 0))


def _row_spec(blk, d):
    return pl.BlockSpec((blk, d), lambda i: (i, 0))


def _lnorm(x, eps=1e-5):
    m = jnp.mean(x, axis=-1, keepdims=True)
    v = jnp.mean((x - m) ** 2, axis=-1, keepdims=True)
    return (x - m) * lax.rsqrt(v + eps)


def _bd(w):
    z = jnp.zeros_like(w)
    return jnp.concatenate(
        [jnp.concatenate([w, z], 1), jnp.concatenate([z, w], 1)], 0)


def _dup(b):
    return jnp.concatenate([b, b]).reshape(1, -1)


def _enc_body(oe, oo, w1, b1, g1, be1, w2, b2, out):
    def half(o):
        h = jnp.dot(o[...], w1[...], preferred_element_type=jnp.float32)
        h = _lnorm(h + b1[...]) * g1[...] + be1[...]
        h = jnp.maximum(h, 0.0)
        h2 = jnp.dot(h, w2[...], preferred_element_type=jnp.float32)
        return jnp.maximum(h2 + b2[...], 0.0)
    out[...] = jnp.concatenate([half(oe), half(oo)], axis=-1)


def _encoder(obs_e, obs_o, p):
    return pl.pallas_call(
        _enc_body,
        grid=(_HALF // _BLK_N,),
        in_specs=[
            _row_spec(_BLK_N, 10), _row_spec(_BLK_N, 10),
            _full_spec(10, _HID), _full_spec(1, _HID),
            _full_spec(1, _HID), _full_spec(1, _HID),
            _full_spec(_HID, _EMB), _full_spec(1, _EMB),
        ],
        out_specs=_row_spec(_BLK_N, 2 * _EMB),
        out_shape=jax.ShapeDtypeStruct((_HALF, 2 * _EMB), jnp.float32),
    )(obs_e, obs_o, p['enc_w1'], p['enc_b1'].reshape(1, -1),
      p['enc_g1'].reshape(1, -1), p['enc_be1'].reshape(1, -1),
      p['enc_w2'], p['enc_b2'].reshape(1, -1))


def _edge_body(cat, eaT, wij, we, b1, w2, b2, out):
    pre = jnp.dot(cat[...], wij[...], preferred_element_type=jnp.float32)
    pre = pre + lax.dot_general(
        eaT[...], we[...], (((0,), (0,)), ((), ())),
        preferred_element_type=jnp.float32)
    h1 = jnp.maximum(pre + b1[...], 0.0)
    msg = jnp.dot(h1, w2[...], preferred_element_type=jnp.float32)
    msg = 0.25 * (msg + b2[...])
    out[...] = jnp.concatenate(
        [msg, jnp.zeros((_BLK_E, _EMB), jnp.float32)], axis=-1)


def _edge_mlp(cat, eaT, wij, we, b1, w2, b2):
    return pl.pallas_call(
        _edge_body,
        grid=(_E // _BLK_E,),
        in_specs=[
            _row_spec(_BLK_E, 2 * _EMB),
            pl.BlockSpec((_EDGE, _BLK_E), lambda i: (0, i)),
            _full_spec(2 * _EMB, _HID), _full_spec(_EDGE, _HID),
            _full_spec(1, _HID),
            _full_spec(_HID, _EMB), _full_spec(1, _EMB),
        ],
        out_specs=_row_spec(_BLK_E, 2 * _EMB),
        out_shape=jax.ShapeDtypeStruct((_E, 2 * _EMB), jnp.float32),
    )(cat, eaT, wij, we, b1.reshape(1, -1), w2, b2.reshape(1, -1))


def _upd_body(h, a, w1h, w1a, b1, w2, b2, g, b, out):
    u = jnp.dot(h[...], w1h[...], preferred_element_type=jnp.float32)
    u = u + jnp.dot(a[...], w1a[...], preferred_element_type=jnp.float32)
    u = jnp.maximum(u + b1[...], 0.0)
    upd = jnp.dot(u, w2[...], preferred_element_type=jnp.float32) + b2[...]
    y = h[...] + upd
    yl = jnp.concatenate(
        [_lnorm(y[:, :_EMB]), _lnorm(y[:, _EMB:])], axis=-1)
    out[...] = yl * g[...] + b[...]


def _update(h_p, aggr_p, bw1h, bw1a, b1, bw2, b2, g, b):
    return pl.pallas_call(
        _upd_body,
        grid=(_HALF // _BLK_N,),
        in_specs=[
            _row_spec(_BLK_N, 2 * _EMB), _row_spec(_BLK_N, 2 * _EMB),
            _full_spec(2 * _EMB, 2 * _HID), _full_spec(2 * _EMB, 2 * _HID),
            _full_spec(1, 2 * _HID),
            _full_spec(2 * _HID, 2 * _EMB), _full_spec(1, 2 * _EMB),
            _full_spec(1, 2 * _EMB), _full_spec(1, 2 * _EMB),
        ],
        out_specs=_row_spec(_BLK_N, 2 * _EMB),
        out_shape=jax.ShapeDtypeStruct((_HALF, 2 * _EMB), jnp.float32),
    )(h_p, aggr_p, bw1h, bw1a, b1, bw2, b2, g, b)


def _out_body(h, w1, b1, w2, b2, out):
    u = jnp.dot(h[...], w1[...], preferred_element_type=jnp.float32)
    u = jnp.maximum(u + b1[...], 0.0)
    out[...] = jnp.dot(u, w2[...], preferred_element_type=jnp.float32) + b2[...]


def _output(h_p, bw1, b1, bw2, b2):
    return pl.pallas_call(
        _out_body,
        grid=(_HALF // _BLK_N,),
        in_specs=[
            _row_spec(_BLK_N, 2 * _EMB),
            _full_spec(2 * _EMB, 2 * _HID), _full_spec(1, 2 * _HID),
            _full_spec(2 * _HID, 2 * _EMB), _full_spec(1, 2 * _EMB),
        ],
        out_specs=_row_spec(_BLK_N, 2 * _EMB),
        out_shape=jax.ShapeDtypeStruct((_HALF, 2 * _EMB), jnp.float32),
    )(h_p, bw1, b1, bw2, b2)


def kernel(obs, edge_index, edge_attr, params):
    p = params
    src = edge_index[0]
    dst = edge_index[1]
    obs_e = obs[0::2, :10]
    obs_o = obs[1::2, :10]
    eaT = edge_attr.T

    h_p = _encoder(obs_e, obs_o, p)
    for lp in p['layers']:
        cat = _gather(h_p.reshape(_N, _EMB), src, dst)
        w1 = lp['msg_w1']
        wmsg = _edge_mlp(cat, eaT, w1[:2 * _EMB], w1[2 * _EMB:],
                         lp['msg_b1'], lp['msg_w2'], lp['msg_b2'])
        aggr_p = _scatter(wmsg, dst)
        uw1 = lp['upd_w1']
        h_p = _update(h_p, aggr_p,
                      _bd(uw1[:_EMB]), _bd(uw1[_EMB:]),
                      _dup(lp['upd_b1']), _bd(lp['upd_w2']),
                      _dup(lp['upd_b2']), _dup(lp['ln_g']),
                      _dup(lp['ln_b']))
    out_p = _output(h_p, _bd(p['out_w1']), _dup(p['out_b1']),
                    _bd(p['out_w2']), _dup(p['out_b2']))
    return out_p.reshape(_N, _EMB)


# speedup vs baseline: 4.8311x; 1.0689x over previous
import functools

import jax
import jax.numpy as jnp
from jax import lax
from jax.experimental import pallas as pl
from jax.experimental.pallas import tpu as pltpu
from jax.experimental.pallas import tpu_sc as plsc

_N = 50000
_E = 800000
_EMB = 64
_HID = 128
_EDGE = 8

_NC = 2
_NS = 16
_NW = _NC * _NS

_GCHUNK = 1024
_G_PER_W = _E // _NW
_G_FULL = _G_PER_W // _GCHUNK
_G_TAIL = _G_PER_W - _G_FULL * _GCHUNK

_SCHUNK = 256
_S_PER_W = _E // _NS
_S_FULL = _S_PER_W // _SCHUNK
_S_TAIL = _S_PER_W - _S_FULL * _SCHUNK

_HALF = _N // 2
_QUART = _HALF // 2
_ODD_BASE = 12800
_PAD_BASE = 25300
_ACC_ROWS = 25600

_BLK_N = 5000
_BLK_E = 3200


def _sc_mesh():
    return plsc.VectorSubcoreMesh(
        core_axis_name="c", subcore_axis_name="s",
        num_cores=_NC, num_subcores=_NS)


def _gather(h64, src, dst):
    @functools.partial(
        pl.kernel,
        out_type=jax.ShapeDtypeStruct((_E, 2 * _EMB), jnp.float32),
        mesh=_sc_mesh(),
        scratch_types=[
            pltpu.VMEM((_GCHUNK,), jnp.int32),
            pltpu.VMEM((_GCHUNK, _EMB), jnp.float32),
            pltpu.SemaphoreType.DMA,
        ],
        compiler_params=pltpu.CompilerParams(use_tc_tiling_on_sc=False),
    )
    def k(h_hbm, src_hbm, dst_hbm, cat_hbm, idx_v, rows_v, sem):
        c = lax.axis_index("c")
        s = lax.axis_index("s")
        wid = s * _NC + c
        base = wid * _G_PER_W

        def one(idx_hbm, col, off, n, nidx):
            pltpu.sync_copy(idx_hbm.at[pl.ds(off, n)], idx_v.at[pl.ds(0, n)])
            cps = [pltpu.async_copy(
                h_hbm.at[idx_v.at[pl.ds(st, ln)]],
                rows_v.at[pl.ds(st, ln)], sem) for (st, ln) in nidx]
            for cp in cps:
                cp.wait()
            pltpu.sync_copy(rows_v.at[pl.ds(0, n)],
                            cat_hbm.at[pl.ds(off, n), pl.ds(col, _EMB)])

        full_slices = [(j * 128, 128) for j in range(_GCHUNK // 128)]
        tail_slices = [(j * 128, 128) for j in range(_G_TAIL // 128)]
        if _G_TAIL % 128:
            tail_slices.append((_G_TAIL - _G_TAIL % 128, _G_TAIL % 128))

        def step(i, carry):
            off = base + i * _GCHUNK
            one(dst_hbm, 0, off, _GCHUNK, full_slices)
            one(src_hbm, _EMB, off, _GCHUNK, full_slices)
            return carry

        lax.fori_loop(0, _G_FULL, step, 0)
        toff = base + _G_FULL * _GCHUNK
        one(dst_hbm, 0, toff, _G_TAIL, tail_slices)
        one(src_hbm, _EMB, toff, _G_TAIL, tail_slices)

    return k(h64, src, dst)


def _scatter(wmsg128, dst):
    @functools.partial(
        pl.kernel,
        out_type=jax.ShapeDtypeStruct((_HALF, 2 * _EMB), jnp.float32),
        mesh=_sc_mesh(),
        scratch_types=[
            pltpu.VMEM((_SCHUNK,), jnp.int32),
            pltpu.VMEM((_SCHUNK // 128, 128), jnp.int32),
            pltpu.VMEM((_SCHUNK, _EMB), jnp.float32),
            pltpu.VMEM_SHARED((_ACC_ROWS, _EMB), jnp.float32),
        ],
        compiler_params=pltpu.CompilerParams(use_tc_tiling_on_sc=False),
    )
    def k(w_hbm, d_hbm, out_hbm, raw_v, idx2_v, vals_v, acc):
        c = lax.axis_index("c")
        s = lax.axis_index("s")
        lane = lax.iota(jnp.int32, 16)
        base = s * _S_PER_W
        nodes0 = c * _HALF
        zb = s * (_ACC_ROWS // _NS)

        def fixup(kv, v):
            local = v - nodes0
            inr = (local >= 0) & (local < _HALF)
            lrow = (local >> 1) + (local & 1) * _ODD_BASE
            pad = _PAD_BASE + s * 16 + ((lane + kv) & 15)
            idx2_v[kv // 8, pl.ds((kv % 8) * 16, 16)] = (
                jnp.where(inr, lrow, pad))

        def zrow(r, carry):
            for t in range(_EMB // 16):
                vals_v[r, pl.ds(t * 16, 16)] = jnp.zeros((16,), jnp.float32)
            return carry
        lax.fori_loop(0, _SCHUNK, zrow, 0)
        for t in range(_ACC_ROWS // _NS // _SCHUNK):
            pltpu.sync_copy(vals_v, acc.at[pl.ds(zb + t * _SCHUNK, _SCHUNK)])
        rem = (_ACC_ROWS // _NS) % _SCHUNK
        if rem:
            pltpu.sync_copy(
                vals_v.at[pl.ds(0, rem)],
                acc.at[pl.ds(zb + (_ACC_ROWS // _NS) - rem, rem)])
        plsc.subcore_barrier()

        def step(i, carry):
            off = base + i * _SCHUNK
            pltpu.sync_copy(d_hbm.at[pl.ds(off, _SCHUNK)], raw_v)
            pltpu.sync_copy(
                w_hbm.at[pl.ds(off, _SCHUNK), pl.ds(0, _EMB)], vals_v)
            for kv in range(_SCHUNK // 16):
                fixup(kv, raw_v[pl.ds(kv * 16, 16)])
            for j in range(_SCHUNK // 128):
                pltpu.sync_copy(vals_v.at[pl.ds(j * 128, 128)],
                                acc.at[idx2_v.at[j]], add=True)
            return carry

        lax.fori_loop(0, _S_FULL, step, 0)
        toff = base + _S_FULL * _SCHUNK
        pltpu.sync_copy(d_hbm.at[pl.ds(toff, _S_TAIL)],
                        raw_v.at[pl.ds(0, _S_TAIL)])
        pltpu.sync_copy(
            w_hbm.at[pl.ds(toff, _S_TAIL), pl.ds(0, _EMB)],
            vals_v.at[pl.ds(0, _S_TAIL)])
        for kv in range(_S_TAIL // 16):
            fixup(kv, raw_v[pl.ds(kv * 16, 16)])
        for kv in range(_S_TAIL // 16, _SCHUNK // 16):
            pad = _PAD_BASE + s * 16 + ((lane + kv) & 15)
            idx2_v[kv // 8, pl.ds((kv % 8) * 16, 16)] = pad
        for j in range(_SCHUNK // 128):
            pltpu.sync_copy(vals_v.at[pl.ds(j * 128, 128)],
                            acc.at[idx2_v.at[j]], add=True)
        plsc.subcore_barrier()

        rows = _QUART // 4
        @pl.when(s < 4)
        def _():
            pltpu.sync_copy(
                acc.at[pl.ds(s * rows, rows)],
                out_hbm.at[pl.ds(c * _QUART + s * rows, rows),
                           pl.ds(0, _EMB)])

        @pl.when((s >= 4) & (s < 8))
        def _():
            pltpu.sync_copy(
                acc.at[pl.ds(_ODD_BASE + (s - 4) * rows, rows)],
                out_hbm.at[pl.ds(c * _QUART + (s - 4) * rows, rows),
                           pl.ds(_EMB, _EMB)])

    return k(wmsg128, dst)


def _full_spec(d1, d2):
    return pl.BlockSpec((d1, d2), lambda i: (0, 0))


def _row_spec(blk, d):
    return pl.BlockSpec((blk, d), lambda i: (i, 0))


def _lnorm(x, eps=1e-5):
    m = jnp.mean(x, axis=-1, keepdims=True)
    v = jnp.mean((x - m) ** 2, axis=-1, keepdims=True)
    return (x - m) * lax.rsqrt(v + eps)


def _bd(w):
    z = jnp.zeros_like(w)
    return jnp.concatenate(
        [jnp.concatenate([w, z], 1), jnp.concatenate([z, w], 1)], 0)


def _dup(b):
    return jnp.concatenate([b, b]).reshape(1, -1)


def _enc_body(oe, oo, w1, b1, g1, be1, w2, b2, out):
    def half(o):
        h = jnp.dot(o[...], w1[...], preferred_element_type=jnp.float32)
        h = _lnorm(h + b1[...]) * g1[...] + be1[...]
        h = jnp.maximum(h, 0.0)
        h2 = jnp.dot(h, w2[...], preferred_element_type=jnp.float32)
        return jnp.maximum(h2 + b2[...], 0.0)
    out[...] = jnp.concatenate([half(oe), half(oo)], axis=-1)


def _encoder(obs_e, obs_o, p):
    return pl.pallas_call(
        _enc_body,
        grid=(_HALF // _BLK_N,),
        in_specs=[
            _row_spec(_BLK_N, 10), _row_spec(_BLK_N, 10),
            _full_spec(10, _HID), _full_spec(1, _HID),
            _full_spec(1, _HID), _full_spec(1, _HID),
            _full_spec(_HID, _EMB), _full_spec(1, _EMB),
        ],
        out_specs=_row_spec(_BLK_N, 2 * _EMB),
        out_shape=jax.ShapeDtypeStruct((_HALF, 2 * _EMB), jnp.float32),
    )(obs_e, obs_o, p['enc_w1'], p['enc_b1'].reshape(1, -1),
      p['enc_g1'].reshape(1, -1), p['enc_be1'].reshape(1, -1),
      p['enc_w2'], p['enc_b2'].reshape(1, -1))


def _edge_body(cat, eaT, wij, we, b1, w2, b2, out):
    pre = jnp.dot(cat[...], wij[...], preferred_element_type=jnp.float32)
    pre = pre + lax.dot_general(
        eaT[...], we[...], (((0,), (0,)), ((), ())),
        preferred_element_type=jnp.float32)
    h1 = jnp.maximum(pre + b1[...], 0.0)
    msg = jnp.dot(h1, w2[...], preferred_element_type=jnp.float32)
    msg = 0.25 * (msg + b2[...])
    out[...] = jnp.concatenate(
        [msg, jnp.zeros((_BLK_E, _EMB), jnp.float32)], axis=-1)


def _edge_mlp(cat, eaT, wij, we, b1, w2, b2):
    return pl.pallas_call(
        _edge_body,
        grid=(_E // _BLK_E,),
        in_specs=[
            _row_spec(_BLK_E, 2 * _EMB),
            pl.BlockSpec((_EDGE, _BLK_E), lambda i: (0, i)),
            _full_spec(2 * _EMB, _HID), _full_spec(_EDGE, _HID),
            _full_spec(1, _HID),
            _full_spec(_HID, _EMB), _full_spec(1, _EMB),
        ],
        out_specs=_row_spec(_BLK_E, 2 * _EMB),
        out_shape=jax.ShapeDtypeStruct((_E, 2 * _EMB), jnp.float32),
    )(cat, eaT, wij, we, b1.reshape(1, -1), w2, b2.reshape(1, -1))


def _upd_body(h, a, w1h, w1a, b1, w2, b2, g, b, out):
    u = jnp.dot(h[...], w1h[...], preferred_element_type=jnp.float32)
    u = u + jnp.dot(a[...], w1a[...], preferred_element_type=jnp.float32)
    u = jnp.maximum(u + b1[...], 0.0)
    upd = jnp.dot(u, w2[...], preferred_element_type=jnp.float32) + b2[...]
    y = h[...] + upd
    yl = jnp.concatenate(
        [_lnorm(y[:, :_EMB]), _lnorm(y[:, _EMB:])], axis=-1)
    out[...] = yl * g[...] + b[...]


def _update(h_p, aggr_p, bw1h, bw1a, b1, bw2, b2, g, b):
    return pl.pallas_call(
        _upd_body,
        grid=(_HALF // _BLK_N,),
        in_specs=[
            _row_spec(_BLK_N, 2 * _EMB), _row_spec(_BLK_N, 2 * _EMB),
            _full_spec(2 * _EMB, 2 * _HID), _full_spec(2 * _EMB, 2 * _HID),
            _full_spec(1, 2 * _HID),
            _full_spec(2 * _HID, 2 * _EMB), _full_spec(1, 2 * _EMB),
            _full_spec(1, 2 * _EMB), _full_spec(1, 2 * _EMB),
        ],
        out_specs=_row_spec(_BLK_N, 2 * _EMB),
        out_shape=jax.ShapeDtypeStruct((_HALF, 2 * _EMB), jnp.float32),
    )(h_p, aggr_p, bw1h, bw1a, b1, bw2, b2, g, b)


def _out_body(h, w1, b1, w2, b2, out):
    u = jnp.dot(h[...], w1[...], preferred_element_type=jnp.float32)
    u = jnp.maximum(u + b1[...], 0.0)
    out[...] = jnp.dot(u, w2[...], preferred_element_type=jnp.float32) + b2[...]


def _output(h_p, bw1, b1, bw2, b2):
    return pl.pallas_call(
        _out_body,
        grid=(_HALF // _BLK_N,),
        in_specs=[
            _row_spec(_BLK_N, 2 * _EMB),
            _full_spec(2 * _EMB, 2 * _HID), _full_spec(1, 2 * _HID),
            _full_spec(2 * _HID, 2 * _EMB), _full_spec(1, 2 * _EMB),
        ],
        out_specs=_row_spec(_BLK_N, 2 * _EMB),
        out_shape=jax.ShapeDtypeStruct((_HALF, 2 * _EMB), jnp.float32),
    )(h_p, bw1, b1, bw2, b2)


def kernel(obs, edge_index, edge_attr, params):
    p = params
    src = edge_index[0]
    dst = edge_index[1]
    obs_e = obs[0::2, :10]
    obs_o = obs[1::2, :10]
    eaT = edge_attr.T

    h_p = _encoder(obs_e, obs_o, p)
    for lp in p['layers']:
        cat = _gather(h_p.reshape(_N, _EMB), src, dst)
        w1 = lp['msg_w1']
        wmsg = _edge_mlp(cat, eaT, w1[:2 * _EMB], w1[2 * _EMB:],
                         lp['msg_b1'], lp['msg_w2'], lp['msg_b2'])
        aggr_p = _scatter(wmsg, dst)
        uw1 = lp['upd_w1']
        h_p = _update(h_p, aggr_p,
                      _bd(uw1[:_EMB]), _bd(uw1[_EMB:]),
                      _dup(lp['upd_b1']), _bd(lp['upd_w2']),
                      _dup(lp['upd_b2']), _dup(lp['ln_g']),
                      _dup(lp['ln_b']))
    out_p = _output(h_p, _bd(p['out_w1']), _dup(p['out_b1']),
                    _bd(p['out_w2']), _dup(p['out_b2']))
    return out_p.reshape(_N, _EMB)


# speedup vs baseline: 4.8335x; 1.0005x over previous
import functools

import jax
import jax.numpy as jnp
from jax import lax
from jax.experimental import pallas as pl
from jax.experimental.pallas import tpu as pltpu
from jax.experimental.pallas import tpu_sc as plsc

_N = 50000
_E = 800000
_EMB = 64
_HID = 128
_EDGE = 8

_NC = 2
_NS = 16
_NW = _NC * _NS

_GCHUNK = 512
_G_PER_W = _E // _NW
_G_FULL = _G_PER_W // _GCHUNK
_G_TAIL = _G_PER_W - _G_FULL * _GCHUNK

_SCHUNK = 256
_S_PER_W = _E // _NS
_S_FULL = _S_PER_W // _SCHUNK
_S_TAIL = _S_PER_W - _S_FULL * _SCHUNK

_HALF = _N // 2
_QUART = _HALF // 2
_ODD_BASE = 12800
_PAD_BASE = 25300
_ACC_ROWS = 25600

_BLK_N = 5000
_BLK_E = 3200


def _sc_mesh():
    return plsc.VectorSubcoreMesh(
        core_axis_name="c", subcore_axis_name="s",
        num_cores=_NC, num_subcores=_NS)


def _gather(h64, src, dst):
    @functools.partial(
        pl.kernel,
        out_type=jax.ShapeDtypeStruct((_E, 2 * _EMB), jnp.float32),
        mesh=_sc_mesh(),
        scratch_types=[
            pltpu.VMEM((_GCHUNK,), jnp.int32),
            pltpu.VMEM((_GCHUNK,), jnp.int32),
            pltpu.VMEM((_GCHUNK, _EMB), jnp.float32),
            pltpu.VMEM((_GCHUNK, _EMB), jnp.float32),
            pltpu.SemaphoreType.DMA,
            pltpu.SemaphoreType.DMA,
            pltpu.SemaphoreType.DMA,
        ],
        compiler_params=pltpu.CompilerParams(use_tc_tiling_on_sc=False),
    )
    def k(h_hbm, src_hbm, dst_hbm, cat_hbm,
          idx_a, idx_b, rows_a, rows_b, sem_i, sem_g, sem_o):
        c = lax.axis_index("c")
        s = lax.axis_index("s")
        wid = s * _NC + c
        base = wid * _G_PER_W

        full_slices = [(j * 128, 128) for j in range(_GCHUNK // 128)]
        tail_slices = [(j * 128, 128) for j in range(_G_TAIL // 128)]
        if _G_TAIL % 128:
            tail_slices.append((_G_TAIL - _G_TAIL % 128, _G_TAIL % 128))

        def out_copy(rows_v, off, col, n):
            return (rows_v.at[pl.ds(0, n)],
                    cat_hbm.at[pl.ds(off, n), pl.ds(col, _EMB)])

        def step(i, carry):
            off = base + i * _GCHUNK

            @pl.when(i > 0)
            def _():
                sa, da = out_copy(rows_a, off, 0, _GCHUNK)
                pltpu.make_async_copy(sa, da, sem_o).wait()
                sb, db = out_copy(rows_b, off, _EMB, _GCHUNK)
                pltpu.make_async_copy(sb, db, sem_o).wait()

            ca = pltpu.async_copy(dst_hbm.at[pl.ds(off, _GCHUNK)],
                                  idx_a, sem_i)
            cb = pltpu.async_copy(src_hbm.at[pl.ds(off, _GCHUNK)],
                                  idx_b, sem_i)
            ca.wait()
            cb.wait()
            cps = [pltpu.async_copy(
                h_hbm.at[idx_a.at[pl.ds(st, ln)]],
                rows_a.at[pl.ds(st, ln)], sem_g)
                for (st, ln) in full_slices]
            cps += [pltpu.async_copy(
                h_hbm.at[idx_b.at[pl.ds(st, ln)]],
                rows_b.at[pl.ds(st, ln)], sem_g)
                for (st, ln) in full_slices]
            for cp in cps:
                cp.wait()
            sa, da = out_copy(rows_a, off, 0, _GCHUNK)
            pltpu.async_copy(sa, da, sem_o)
            sb, db = out_copy(rows_b, off, _EMB, _GCHUNK)
            pltpu.async_copy(sb, db, sem_o)
            return carry

        lax.fori_loop(0, _G_FULL, step, 0)
        toff = base + _G_FULL * _GCHUNK
        sa, da = out_copy(rows_a, base, 0, _GCHUNK)
        pltpu.make_async_copy(sa, da, sem_o).wait()
        sb, db = out_copy(rows_b, base, _EMB, _GCHUNK)
        pltpu.make_async_copy(sb, db, sem_o).wait()

        def one(idx_hbm, idx_v, rows_v, col):
            pltpu.sync_copy(idx_hbm.at[pl.ds(toff, _G_TAIL)],
                            idx_v.at[pl.ds(0, _G_TAIL)])
            cps = [pltpu.async_copy(
                h_hbm.at[idx_v.at[pl.ds(st, ln)]],
                rows_v.at[pl.ds(st, ln)], sem_g) for (st, ln) in tail_slices]
            for cp in cps:
                cp.wait()
            pltpu.sync_copy(rows_v.at[pl.ds(0, _G_TAIL)],
                            cat_hbm.at[pl.ds(toff, _G_TAIL),
                                       pl.ds(col, _EMB)])
        one(dst_hbm, idx_a, rows_a, 0)
        one(src_hbm, idx_b, rows_b, _EMB)

    return k(h64, src, dst)


def _scatter(wmsg128, dst):
    @functools.partial(
        pl.kernel,
        out_type=jax.ShapeDtypeStruct((_HALF, 2 * _EMB), jnp.float32),
        mesh=_sc_mesh(),
        scratch_types=[
            pltpu.VMEM((_SCHUNK,), jnp.int32),
            pltpu.VMEM((_SCHUNK // 128, 128), jnp.int32),
            pltpu.VMEM((_SCHUNK, _EMB), jnp.float32),
            pltpu.VMEM_SHARED((_ACC_ROWS, _EMB), jnp.float32),
        ],
        compiler_params=pltpu.CompilerParams(use_tc_tiling_on_sc=False),
    )
    def k(w_hbm, d_hbm, out_hbm, raw_v, idx2_v, vals_v, acc):
        c = lax.axis_index("c")
        s = lax.axis_index("s")
        lane = lax.iota(jnp.int32, 16)
        base = s * _S_PER_W
        nodes0 = c * _HALF
        zb = s * (_ACC_ROWS // _NS)

        def fixup(kv, v):
            local = v - nodes0
            inr = (local >= 0) & (local < _HALF)
            lrow = (local >> 1) + (local & 1) * _ODD_BASE
            pad = _PAD_BASE + s * 16 + ((lane + kv) & 15)
            idx2_v[kv // 8, pl.ds((kv % 8) * 16, 16)] = (
                jnp.where(inr, lrow, pad))

        def zrow(r, carry):
            for t in range(_EMB // 16):
                vals_v[r, pl.ds(t * 16, 16)] = jnp.zeros((16,), jnp.float32)
            return carry
        lax.fori_loop(0, _SCHUNK, zrow, 0)
        for t in range(_ACC_ROWS // _NS // _SCHUNK):
            pltpu.sync_copy(vals_v, acc.at[pl.ds(zb + t * _SCHUNK, _SCHUNK)])
        rem = (_ACC_ROWS // _NS) % _SCHUNK
        if rem:
            pltpu.sync_copy(
                vals_v.at[pl.ds(0, rem)],
                acc.at[pl.ds(zb + (_ACC_ROWS // _NS) - rem, rem)])
        plsc.subcore_barrier()

        def step(i, carry):
            off = base + i * _SCHUNK
            pltpu.sync_copy(d_hbm.at[pl.ds(off, _SCHUNK)], raw_v)
            pltpu.sync_copy(
                w_hbm.at[pl.ds(off, _SCHUNK), pl.ds(0, _EMB)], vals_v)
            for kv in range(_SCHUNK // 16):
                fixup(kv, raw_v[pl.ds(kv * 16, 16)])
            for j in range(_SCHUNK // 128):
                pltpu.sync_copy(vals_v.at[pl.ds(j * 128, 128)],
                                acc.at[idx2_v.at[j]], add=True)
            return carry

        lax.fori_loop(0, _S_FULL, step, 0)
        toff = base + _S_FULL * _SCHUNK
        pltpu.sync_copy(d_hbm.at[pl.ds(toff, _S_TAIL)],
                        raw_v.at[pl.ds(0, _S_TAIL)])
        pltpu.sync_copy(
            w_hbm.at[pl.ds(toff, _S_TAIL), pl.ds(0, _EMB)],
            vals_v.at[pl.ds(0, _S_TAIL)])
        for kv in range(_S_TAIL // 16):
            fixup(kv, raw_v[pl.ds(kv * 16, 16)])
        for kv in range(_S_TAIL // 16, _SCHUNK // 16):
            pad = _PAD_BASE + s * 16 + ((lane + kv) & 15)
            idx2_v[kv // 8, pl.ds((kv % 8) * 16, 16)] = pad
        for j in range(_SCHUNK // 128):
            pltpu.sync_copy(vals_v.at[pl.ds(j * 128, 128)],
                            acc.at[idx2_v.at[j]], add=True)
        plsc.subcore_barrier()

        rows = _QUART // 4
        @pl.when(s < 4)
        def _():
            pltpu.sync_copy(
                acc.at[pl.ds(s * rows, rows)],
                out_hbm.at[pl.ds(c * _QUART + s * rows, rows),
                           pl.ds(0, _EMB)])

        @pl.when((s >= 4) & (s < 8))
        def _():
            pltpu.sync_copy(
                acc.at[pl.ds(_ODD_BASE + (s - 4) * rows, rows)],
                out_hbm.at[pl.ds(c * _QUART + (s - 4) * rows, rows),
                           pl.ds(_EMB, _EMB)])

    return k(wmsg128, dst)


def _full_spec(d1, d2):
    return pl.BlockSpec((d1, d2), lambda i: (0, 0))


def _row_spec(blk, d):
    return pl.BlockSpec((blk, d), lambda i: (i, 0))


def _lnorm(x, eps=1e-5):
    m = jnp.mean(x, axis=-1, keepdims=True)
    v = jnp.mean((x - m) ** 2, axis=-1, keepdims=True)
    return (x - m) * lax.rsqrt(v + eps)


def _bd(w):
    z = jnp.zeros_like(w)
    return jnp.concatenate(
        [jnp.concatenate([w, z], 1), jnp.concatenate([z, w], 1)], 0)


def _dup(b):
    return jnp.concatenate([b, b]).reshape(1, -1)


def _enc_body(oe, oo, w1, b1, g1, be1, w2, b2, out):
    def half(o):
        h = jnp.dot(o[...], w1[...], preferred_element_type=jnp.float32)
        h = _lnorm(h + b1[...]) * g1[...] + be1[...]
        h = jnp.maximum(h, 0.0)
        h2 = jnp.dot(h, w2[...], preferred_element_type=jnp.float32)
        return jnp.maximum(h2 + b2[...], 0.0)
    out[...] = jnp.concatenate([half(oe), half(oo)], axis=-1)


def _encoder(obs_e, obs_o, p):
    return pl.pallas_call(
        _enc_body,
        grid=(_HALF // _BLK_N,),
        in_specs=[
            _row_spec(_BLK_N, 10), _row_spec(_BLK_N, 10),
            _full_spec(10, _HID), _full_spec(1, _HID),
            _full_spec(1, _HID), _full_spec(1, _HID),
            _full_spec(_HID, _EMB), _full_spec(1, _EMB),
        ],
        out_specs=_row_spec(_BLK_N, 2 * _EMB),
        out_shape=jax.ShapeDtypeStruct((_HALF, 2 * _EMB), jnp.float32),
    )(obs_e, obs_o, p['enc_w1'], p['enc_b1'].reshape(1, -1),
      p['enc_g1'].reshape(1, -1), p['enc_be1'].reshape(1, -1),
      p['enc_w2'], p['enc_b2'].reshape(1, -1))


def _edge_body(cat, eaT, wij, we, b1, w2, b2, out):
    pre = jnp.dot(cat[...], wij[...], preferred_element_type=jnp.float32)
    pre = pre + lax.dot_general(
        eaT[...], we[...], (((0,), (0,)), ((), ())),
        preferred_element_type=jnp.float32)
    h1 = jnp.maximum(pre + b1[...], 0.0)
    msg = jnp.dot(h1, w2[...], preferred_element_type=jnp.float32)
    msg = 0.25 * (msg + b2[...])
    out[...] = jnp.concatenate(
        [msg, jnp.zeros((_BLK_E, _EMB), jnp.float32)], axis=-1)


def _edge_mlp(cat, eaT, wij, we, b1, w2, b2):
    return pl.pallas_call(
        _edge_body,
        grid=(_E // _BLK_E,),
        in_specs=[
            _row_spec(_BLK_E, 2 * _EMB),
            pl.BlockSpec((_EDGE, _BLK_E), lambda i: (0, i)),
            _full_spec(2 * _EMB, _HID), _full_spec(_EDGE, _HID),
            _full_spec(1, _HID),
            _full_spec(_HID, _EMB), _full_spec(1, _EMB),
        ],
        out_specs=_row_spec(_BLK_E, 2 * _EMB),
        out_shape=jax.ShapeDtypeStruct((_E, 2 * _EMB), jnp.float32),
    )(cat, eaT, wij, we, b1.reshape(1, -1), w2, b2.reshape(1, -1))


def _upd_body(h, a, w1h, w1a, b1, w2, b2, g, b, out):
    u = jnp.dot(h[...], w1h[...], preferred_element_type=jnp.float32)
    u = u + jnp.dot(a[...], w1a[...], preferred_element_type=jnp.float32)
    u = jnp.maximum(u + b1[...], 0.0)
    upd = jnp.dot(u, w2[...], preferred_element_type=jnp.float32) + b2[...]
    y = h[...] + upd
    yl = jnp.concatenate(
        [_lnorm(y[:, :_EMB]), _lnorm(y[:, _EMB:])], axis=-1)
    out[...] = yl * g[...] + b[...]


def _update(h_p, aggr_p, bw1h, bw1a, b1, bw2, b2, g, b):
    return pl.pallas_call(
        _upd_body,
        grid=(_HALF // _BLK_N,),
        in_specs=[
            _row_spec(_BLK_N, 2 * _EMB), _row_spec(_BLK_N, 2 * _EMB),
            _full_spec(2 * _EMB, 2 * _HID), _full_spec(2 * _EMB, 2 * _HID),
            _full_spec(1, 2 * _HID),
            _full_spec(2 * _HID, 2 * _EMB), _full_spec(1, 2 * _EMB),
            _full_spec(1, 2 * _EMB), _full_spec(1, 2 * _EMB),
        ],
        out_specs=_row_spec(_BLK_N, 2 * _EMB),
        out_shape=jax.ShapeDtypeStruct((_HALF, 2 * _EMB), jnp.float32),
    )(h_p, aggr_p, bw1h, bw1a, b1, bw2, b2, g, b)


def _out_body(h, w1, b1, w2, b2, out):
    u = jnp.dot(h[...], w1[...], preferred_element_type=jnp.float32)
    u = jnp.maximum(u + b1[...], 0.0)
    out[...] = jnp.dot(u, w2[...], preferred_element_type=jnp.float32) + b2[...]


def _output(h_p, bw1, b1, bw2, b2):
    return pl.pallas_call(
        _out_body,
        grid=(_HALF // _BLK_N,),
        in_specs=[
            _row_spec(_BLK_N, 2 * _EMB),
            _full_spec(2 * _EMB, 2 * _HID), _full_spec(1, 2 * _HID),
            _full_spec(2 * _HID, 2 * _EMB), _full_spec(1, 2 * _EMB),
        ],
        out_specs=_row_spec(_BLK_N, 2 * _EMB),
        out_shape=jax.ShapeDtypeStruct((_HALF, 2 * _EMB), jnp.float32),
    )(h_p, bw1, b1, bw2, b2)


def kernel(obs, edge_index, edge_attr, params):
    p = params
    src = edge_index[0]
    dst = edge_index[1]
    obs_e = obs[0::2, :10]
    obs_o = obs[1::2, :10]
    eaT = edge_attr.T

    h_p = _encoder(obs_e, obs_o, p)
    for lp in p['layers']:
        cat = _gather(h_p.reshape(_N, _EMB), src, dst)
        w1 = lp['msg_w1']
        wmsg = _edge_mlp(cat, eaT, w1[:2 * _EMB], w1[2 * _EMB:],
                         lp['msg_b1'], lp['msg_w2'], lp['msg_b2'])
        aggr_p = _scatter(wmsg, dst)
        uw1 = lp['upd_w1']
        h_p = _update(h_p, aggr_p,
                      _bd(uw1[:_EMB]), _bd(uw1[_EMB:]),
                      _dup(lp['upd_b1']), _bd(lp['upd_w2']),
                      _dup(lp['upd_b2']), _dup(lp['ln_g']),
                      _dup(lp['ln_b']))
    out_p = _output(h_p, _bd(p['out_w1']), _dup(p['out_b1']),
                    _bd(p['out_w2']), _dup(p['out_b2']))
    return out_p.reshape(_N, _EMB)


# speedup vs baseline: 5.0563x; 1.0461x over previous
import functools

import jax
import jax.numpy as jnp
from jax import lax
from jax.experimental import pallas as pl
from jax.experimental.pallas import tpu as pltpu
from jax.experimental.pallas import tpu_sc as plsc

_N = 50000
_E = 800000
_EMB = 64
_HID = 128
_EDGE = 8

_NC = 2
_NS = 16
_NW = _NC * _NS

_EA = 409600
_EB = _E - _EA

_GCHUNK = 512

_HALF = _N // 2
_QUART = _HALF // 2
_ODD_BASE = 12800
_PAD_BASE = 25300
_ACC_ROWS = 25600

_BLK_N = 5000
_BLK_E = 3200


def _sc_mesh():
    return plsc.VectorSubcoreMesh(
        core_axis_name="c", subcore_axis_name="s",
        num_cores=_NC, num_subcores=_NS)


def _gather(h64, src, dst, e0, n):
    per_w = n // _NW
    n_full = per_w // _GCHUNK
    tail = per_w - n_full * _GCHUNK

    @functools.partial(
        pl.kernel,
        out_type=jax.ShapeDtypeStruct((n, 2 * _EMB), jnp.float32),
        mesh=_sc_mesh(),
        scratch_types=[
            pltpu.VMEM((_GCHUNK,), jnp.int32),
            pltpu.VMEM((_GCHUNK,), jnp.int32),
            pltpu.VMEM((_GCHUNK, _EMB), jnp.float32),
            pltpu.VMEM((_GCHUNK, _EMB), jnp.float32),
            pltpu.SemaphoreType.DMA,
            pltpu.SemaphoreType.DMA,
            pltpu.SemaphoreType.DMA,
        ],
        compiler_params=pltpu.CompilerParams(use_tc_tiling_on_sc=False),
    )
    def k(h_hbm, src_hbm, dst_hbm, cat_hbm,
          idx_a, idx_b, rows_a, rows_b, sem_i, sem_g, sem_o):
        c = lax.axis_index("c")
        s = lax.axis_index("s")
        wid = s * _NC + c
        base = wid * per_w

        full_slices = [(j * 128, 128) for j in range(_GCHUNK // 128)]
        tail_slices = [(j * 128, 128) for j in range(tail // 128)]
        if tail % 128:
            tail_slices.append((tail - tail % 128, tail % 128))

        def out_copy(rows_v, off, col, m):
            return (rows_v.at[pl.ds(0, m)],
                    cat_hbm.at[pl.ds(off - e0, m), pl.ds(col, _EMB)])

        def step(i, carry):
            off = e0 + base + i * _GCHUNK

            @pl.when(i > 0)
            def _():
                sa, da = out_copy(rows_a, off, 0, _GCHUNK)
                pltpu.make_async_copy(sa, da, sem_o).wait()
                sb, db = out_copy(rows_b, off, _EMB, _GCHUNK)
                pltpu.make_async_copy(sb, db, sem_o).wait()

            ca = pltpu.async_copy(dst_hbm.at[pl.ds(off, _GCHUNK)],
                                  idx_a, sem_i)
            cb = pltpu.async_copy(src_hbm.at[pl.ds(off, _GCHUNK)],
                                  idx_b, sem_i)
            ca.wait()
            cb.wait()
            cps = [pltpu.async_copy(
                h_hbm.at[idx_a.at[pl.ds(st, ln)]],
                rows_a.at[pl.ds(st, ln)], sem_g)
                for (st, ln) in full_slices]
            cps += [pltpu.async_copy(
                h_hbm.at[idx_b.at[pl.ds(st, ln)]],
                rows_b.at[pl.ds(st, ln)], sem_g)
                for (st, ln) in full_slices]
            for cp in cps:
                cp.wait()
            sa, da = out_copy(rows_a, off, 0, _GCHUNK)
            pltpu.async_copy(sa, da, sem_o)
            sb, db = out_copy(rows_b, off, _EMB, _GCHUNK)
            pltpu.async_copy(sb, db, sem_o)
            return carry

        lax.fori_loop(0, n_full, step, 0)
        sa, da = out_copy(rows_a, e0 + base, 0, _GCHUNK)
        pltpu.make_async_copy(sa, da, sem_o).wait()
        sb, db = out_copy(rows_b, e0 + base, _EMB, _GCHUNK)
        pltpu.make_async_copy(sb, db, sem_o).wait()

        if tail:
            toff = e0 + base + n_full * _GCHUNK

            def one(idx_hbm, idx_v, rows_v, col):
                pltpu.sync_copy(idx_hbm.at[pl.ds(toff, tail)],
                                idx_v.at[pl.ds(0, tail)])
                cps = [pltpu.async_copy(
                    h_hbm.at[idx_v.at[pl.ds(st, ln)]],
                    rows_v.at[pl.ds(st, ln)], sem_g)
                    for (st, ln) in tail_slices]
                for cp in cps:
                    cp.wait()
                pltpu.sync_copy(rows_v.at[pl.ds(0, tail)],
                                cat_hbm.at[pl.ds(toff - e0, tail),
                                           pl.ds(col, _EMB)])
            one(dst_hbm, idx_a, rows_a, 0)
            one(src_hbm, idx_b, rows_b, _EMB)

    return k(h64, src, dst)


def _scatter(wmsg2, dst, e0, n):
    rows_per_s = n // 2 // _NS
    n_full = rows_per_s // 128
    tail = rows_per_s - n_full * 128

    @functools.partial(
        pl.kernel,
        out_type=jax.ShapeDtypeStruct((_HALF, 2 * _EMB), jnp.float32),
        mesh=_sc_mesh(),
        scratch_types=[
            pltpu.VMEM((256,), jnp.int32),
            pltpu.VMEM((2, 128), jnp.int32),
            pltpu.VMEM((128, _EMB), jnp.float32),
            pltpu.VMEM((128, _EMB), jnp.float32),
            pltpu.VMEM_SHARED((_ACC_ROWS, _EMB), jnp.float32),
        ],
        compiler_params=pltpu.CompilerParams(use_tc_tiling_on_sc=False),
    )
    def k(w_hbm, d_hbm, out_hbm, raw_v, idx2_v, vals_v, vals2_v, acc):
        c = lax.axis_index("c")
        s = lax.axis_index("s")
        lane = lax.iota(jnp.int32, 16)
        row_base = s * rows_per_s
        nodes0 = c * _HALF

        def fixup(kv, nvalid):
            v = raw_v[pl.ds(kv * 16, 16)]
            local = v - nodes0
            inr = (local >= 0) & (local < _HALF)
            if nvalid < 16:
                inr = inr & (lane < nvalid)
            lrow = (local >> 1) + (local & 1) * _ODD_BASE
            pad = _PAD_BASE + s * 16 + ((lane + kv) & 15)
            idx2_v[kv // 8, pl.ds((kv % 8) * 16, 16)] = (
                jnp.where(inr, lrow, pad))

        def zrow(r, carry):
            for t in range(_EMB // 16):
                vals_v[r, pl.ds(t * 16, 16)] = jnp.zeros((16,), jnp.float32)
            return carry
        lax.fori_loop(0, 128, zrow, 0)
        zb = s * (_ACC_ROWS // _NS)
        for t in range(_ACC_ROWS // _NS // 128):
            pltpu.sync_copy(vals_v, acc.at[pl.ds(zb + t * 128, 128)])
        rem = (_ACC_ROWS // _NS) % 128
        if rem:
            pltpu.sync_copy(
                vals_v.at[pl.ds(0, rem)],
                acc.at[pl.ds(zb + (_ACC_ROWS // _NS) - rem, rem)])
        plsc.subcore_barrier()

        def chunk(roff, nrows):
            pltpu.sync_copy(w_hbm.at[pl.ds(roff, nrows), pl.ds(0, _EMB)],
                            vals_v.at[pl.ds(0, nrows)])
            pltpu.sync_copy(
                w_hbm.at[pl.ds(roff, nrows), pl.ds(_EMB, _EMB)],
                vals2_v.at[pl.ds(0, nrows)])
            pltpu.sync_copy(d_hbm.at[pl.ds(e0 + roff, nrows)],
                            raw_v.at[pl.ds(0, nrows)])
            pltpu.sync_copy(d_hbm.at[pl.ds(e0 + n // 2 + roff, nrows)],
                            raw_v.at[pl.ds(128, nrows)])
            for half in range(2):
                for kv in range(8):
                    nvalid = max(0, min(16, nrows - kv * 16))
                    fixup(half * 8 + kv, nvalid)
            pltpu.sync_copy(vals_v, acc.at[idx2_v.at[0]], add=True)
            pltpu.sync_copy(vals2_v, acc.at[idx2_v.at[1]], add=True)

        def step(i, carry):
            chunk(row_base + i * 128, 128)
            return carry

        lax.fori_loop(0, n_full, step, 0)
        if tail:
            chunk(row_base + n_full * 128, tail)
        plsc.subcore_barrier()

        rows = _QUART // 4
        @pl.when(s < 4)
        def _():
            pltpu.sync_copy(
                acc.at[pl.ds(s * rows, rows)],
                out_hbm.at[pl.ds(c * _QUART + s * rows, rows),
                           pl.ds(0, _EMB)])

        @pl.when((s >= 4) & (s < 8))
        def _():
            pltpu.sync_copy(
                acc.at[pl.ds(_ODD_BASE + (s - 4) * rows, rows)],
                out_hbm.at[pl.ds(c * _QUART + (s - 4) * rows, rows),
                           pl.ds(_EMB, _EMB)])

    return k(wmsg2, dst)


def _full_spec(d1, d2):
    return pl.BlockSpec((d1, d2), lambda i: (0, 0))


def _row_spec(blk, d):
    return pl.BlockSpec((blk, d), lambda i: (i, 0))


def _lnorm(x, eps=1e-5):
    m = jnp.mean(x, axis=-1, keepdims=True)
    v = jnp.mean((x - m) ** 2, axis=-1, keepdims=True)
    return (x - m) * lax.rsqrt(v + eps)


def _bd(w):
    z = jnp.zeros_like(w)
    return jnp.concatenate(
        [jnp.concatenate([w, z], 1), jnp.concatenate([z, w], 1)], 0)


def _dup(b):
    return jnp.concatenate([b, b]).reshape(1, -1)


def _enc_body(oe, oo, w1, b1, g1, be1, w2, b2, out):
    def half(o):
        h = jnp.dot(o[...], w1[...], preferred_element_type=jnp.float32)
        h = _lnorm(h + b1[...]) * g1[...] + be1[...]
        h = jnp.maximum(h, 0.0)
        h2 = jnp.dot(h, w2[...], preferred_element_type=jnp.float32)
        return jnp.maximum(h2 + b2[...], 0.0)
    out[...] = jnp.concatenate([half(oe), half(oo)], axis=-1)


def _encoder(obs_e, obs_o, p):
    return pl.pallas_call(
        _enc_body,
        grid=(_HALF // _BLK_N,),
        in_specs=[
            _row_spec(_BLK_N, 10), _row_spec(_BLK_N, 10),
            _full_spec(10, _HID), _full_spec(1, _HID),
            _full_spec(1, _HID), _full_spec(1, _HID),
            _full_spec(_HID, _EMB), _full_spec(1, _EMB),
        ],
        out_specs=_row_spec(_BLK_N, 2 * _EMB),
        out_shape=jax.ShapeDtypeStruct((_HALF, 2 * _EMB), jnp.float32),
    )(obs_e, obs_o, p['enc_w1'], p['enc_b1'].reshape(1, -1),
      p['enc_g1'].reshape(1, -1), p['enc_be1'].reshape(1, -1),
      p['enc_w2'], p['enc_b2'].reshape(1, -1))


def _edge_body(cat1, cat2, eaT1, eaT2, wij, we, b1, w2, b2, out):
    def part(cat, eaT):
        pre = jnp.dot(cat[...], wij[...], preferred_element_type=jnp.float32)
        pre = pre + lax.dot_general(
            eaT[...], we[...], (((0,), (0,)), ((), ())),
            preferred_element_type=jnp.float32)
        h1 = jnp.maximum(pre + b1[...], 0.0)
        msg = jnp.dot(h1, w2[...], preferred_element_type=jnp.float32)
        return 0.25 * (msg + b2[...])
    out[...] = jnp.concatenate([part(cat1, eaT1), part(cat2, eaT2)], -1)


def _edge_mlp(cat, eaT, e0, n, wij, we, b1, w2, b2):
    hb = n // 2 // _BLK_E
    ea1 = e0 // _BLK_E
    ea2 = (e0 + n // 2) // _BLK_E
    return pl.pallas_call(
        _edge_body,
        grid=(hb,),
        in_specs=[
            pl.BlockSpec((_BLK_E, 2 * _EMB), lambda i: (i, 0)),
            pl.BlockSpec((_BLK_E, 2 * _EMB), lambda i, _hb=hb: (i + _hb, 0)),
            pl.BlockSpec((_EDGE, _BLK_E), lambda i, _o=ea1: (0, i + _o)),
            pl.BlockSpec((_EDGE, _BLK_E), lambda i, _o=ea2: (0, i + _o)),
            _full_spec(2 * _EMB, _HID), _full_spec(_EDGE, _HID),
            _full_spec(1, _HID),
            _full_spec(_HID, _EMB), _full_spec(1, _EMB),
        ],
        out_specs=_row_spec(_BLK_E, 2 * _EMB),
        out_shape=jax.ShapeDtypeStruct((n // 2, 2 * _EMB), jnp.float32),
    )(cat, cat, eaT, eaT, wij, we, b1.reshape(1, -1), w2, b2.reshape(1, -1))


def _upd_body(h, aa, ab, w1h, w1a, b1, w2, b2, g, b, out):
    u = jnp.dot(h[...], w1h[...], preferred_element_type=jnp.float32)
    u = u + jnp.dot(aa[...] + ab[...], w1a[...],
                    preferred_element_type=jnp.float32)
    u = jnp.maximum(u + b1[...], 0.0)
    upd = jnp.dot(u, w2[...], preferred_element_type=jnp.float32) + b2[...]
    y = h[...] + upd
    yl = jnp.concatenate(
        [_lnorm(y[:, :_EMB]), _lnorm(y[:, _EMB:])], axis=-1)
    out[...] = yl * g[...] + b[...]


def _update(h_p, aggr_a, aggr_b, bw1h, bw1a, b1, bw2, b2, g, b):
    return pl.pallas_call(
        _upd_body,
        grid=(_HALF // _BLK_N,),
        in_specs=[
            _row_spec(_BLK_N, 2 * _EMB), _row_spec(_BLK_N, 2 * _EMB),
            _row_spec(_BLK_N, 2 * _EMB),
            _full_spec(2 * _EMB, 2 * _HID), _full_spec(2 * _EMB, 2 * _HID),
            _full_spec(1, 2 * _HID),
            _full_spec(2 * _HID, 2 * _EMB), _full_spec(1, 2 * _EMB),
            _full_spec(1, 2 * _EMB), _full_spec(1, 2 * _EMB),
        ],
        out_specs=_row_spec(_BLK_N, 2 * _EMB),
        out_shape=jax.ShapeDtypeStruct((_HALF, 2 * _EMB), jnp.float32),
    )(h_p, aggr_a, aggr_b, bw1h, bw1a, b1, bw2, b2, g, b)


def _out_body(h, w1, b1, w2, b2, out):
    u = jnp.dot(h[...], w1[...], preferred_element_type=jnp.float32)
    u = jnp.maximum(u + b1[...], 0.0)
    out[...] = jnp.dot(u, w2[...], preferred_element_type=jnp.float32) + b2[...]


def _output(h_p, bw1, b1, bw2, b2):
    return pl.pallas_call(
        _out_body,
        grid=(_HALF // _BLK_N,),
        in_specs=[
            _row_spec(_BLK_N, 2 * _EMB),
            _full_spec(2 * _EMB, 2 * _HID), _full_spec(1, 2 * _HID),
            _full_spec(2 * _HID, 2 * _EMB), _full_spec(1, 2 * _EMB),
        ],
        out_specs=_row_spec(_BLK_N, 2 * _EMB),
        out_shape=jax.ShapeDtypeStruct((_HALF, 2 * _EMB), jnp.float32),
    )(h_p, bw1, b1, bw2, b2)


def kernel(obs, edge_index, edge_attr, params):
    p = params
    src = edge_index[0]
    dst = edge_index[1]
    obs_e = obs[0::2, :10]
    obs_o = obs[1::2, :10]
    eaT = edge_attr.T

    h_p = _encoder(obs_e, obs_o, p)
    for lp in p['layers']:
        h64 = h_p.reshape(_N, _EMB)
        w1 = lp['msg_w1']
        cat_a = _gather(h64, src, dst, 0, _EA)
        cat_b = _gather(h64, src, dst, _EA, _EB)
        wmsg_a = _edge_mlp(cat_a, eaT, 0, _EA, w1[:2 * _EMB], w1[2 * _EMB:],
                           lp['msg_b1'], lp['msg_w2'], lp['msg_b2'])
        aggr_a = _scatter(wmsg_a, dst, 0, _EA)
        wmsg_b = _edge_mlp(cat_b, eaT, _EA, _EB, w1[:2 * _EMB],
                           w1[2 * _EMB:], lp['msg_b1'], lp['msg_w2'],
                           lp['msg_b2'])
        aggr_b = _scatter(wmsg_b, dst, _EA, _EB)
        uw1 = lp['upd_w1']
        h_p = _update(h_p, aggr_a, aggr_b,
                      _bd(uw1[:_EMB]), _bd(uw1[_EMB:]),
                      _dup(lp['upd_b1']), _bd(lp['upd_w2']),
                      _dup(lp['upd_b2']), _dup(lp['ln_g']),
                      _dup(lp['ln_b']))
    out_p = _output(h_p, _bd(p['out_w1']), _dup(p['out_b1']),
                    _bd(p['out_w2']), _dup(p['out_b2']))
    return out_p.reshape(_N, _EMB)


# speedup vs baseline: 6.5810x; 1.3015x over previous
import functools

import jax
import jax.numpy as jnp
from jax import lax
from jax.experimental import pallas as pl
from jax.experimental.pallas import tpu as pltpu
from jax.experimental.pallas import tpu_sc as plsc

_N = 50000
_E = 800000
_EMB = 64
_HID = 128
_EDGE = 8

_NC = 2
_NS = 16
_NW = _NC * _NS

_EA = 409600
_EB = _E - _EA

_GCHUNK = 512

_HALF = _N // 2
_QUART = _HALF // 2
_ODD_BASE = 12800
_PAD_BASE = 25300
_ACC_ROWS = 25600

_BLK_N = 5000
_BLK_E = 3200


def _sc_mesh():
    return plsc.VectorSubcoreMesh(
        core_axis_name="c", subcore_axis_name="s",
        num_cores=_NC, num_subcores=_NS)


def _gather(h64, src, dst, e0, n):
    per_w = n // _NW
    n_full = per_w // _GCHUNK
    tail = per_w - n_full * _GCHUNK

    @functools.partial(
        pl.kernel,
        out_type=jax.ShapeDtypeStruct((n, 2 * _EMB), jnp.float32),
        mesh=_sc_mesh(),
        scratch_types=[
            pltpu.VMEM((_GCHUNK,), jnp.int32),
            pltpu.VMEM((_GCHUNK,), jnp.int32),
            pltpu.VMEM((_GCHUNK, _EMB), jnp.float32),
            pltpu.VMEM((_GCHUNK, _EMB), jnp.float32),
            pltpu.SemaphoreType.DMA,
            pltpu.SemaphoreType.DMA,
            pltpu.SemaphoreType.DMA,
        ],
        compiler_params=pltpu.CompilerParams(use_tc_tiling_on_sc=False),
    )
    def k(h_hbm, src_hbm, dst_hbm, cat_hbm,
          idx_a, idx_b, rows_a, rows_b, sem_i, sem_g, sem_o):
        c = lax.axis_index("c")
        s = lax.axis_index("s")
        wid = s * _NC + c
        base = wid * per_w

        full_slices = [(j * 128, 128) for j in range(_GCHUNK // 128)]
        tail_slices = [(j * 128, 128) for j in range(tail // 128)]
        if tail % 128:
            tail_slices.append((tail - tail % 128, tail % 128))

        def out_copy(rows_v, off, col, m):
            return (rows_v.at[pl.ds(0, m)],
                    cat_hbm.at[pl.ds(off - e0, m), pl.ds(col, _EMB)])

        def step(i, carry):
            off = e0 + base + i * _GCHUNK

            @pl.when(i > 0)
            def _():
                sa, da = out_copy(rows_a, off, 0, _GCHUNK)
                pltpu.make_async_copy(sa, da, sem_o).wait()
                sb, db = out_copy(rows_b, off, _EMB, _GCHUNK)
                pltpu.make_async_copy(sb, db, sem_o).wait()

            ca = pltpu.async_copy(dst_hbm.at[pl.ds(off, _GCHUNK)],
                                  idx_a, sem_i)
            cb = pltpu.async_copy(src_hbm.at[pl.ds(off, _GCHUNK)],
                                  idx_b, sem_i)
            ca.wait()
            cb.wait()
            cps = [pltpu.async_copy(
                h_hbm.at[idx_a.at[pl.ds(st, ln)]],
                rows_a.at[pl.ds(st, ln)], sem_g)
                for (st, ln) in full_slices]
            cps += [pltpu.async_copy(
                h_hbm.at[idx_b.at[pl.ds(st, ln)]],
                rows_b.at[pl.ds(st, ln)], sem_g)
                for (st, ln) in full_slices]
            for cp in cps:
                cp.wait()
            sa, da = out_copy(rows_a, off, 0, _GCHUNK)
            pltpu.async_copy(sa, da, sem_o)
            sb, db = out_copy(rows_b, off, _EMB, _GCHUNK)
            pltpu.async_copy(sb, db, sem_o)
            return carry

        lax.fori_loop(0, n_full, step, 0)
        sa, da = out_copy(rows_a, e0 + base, 0, _GCHUNK)
        pltpu.make_async_copy(sa, da, sem_o).wait()
        sb, db = out_copy(rows_b, e0 + base, _EMB, _GCHUNK)
        pltpu.make_async_copy(sb, db, sem_o).wait()

        if tail:
            toff = e0 + base + n_full * _GCHUNK

            def one(idx_hbm, idx_v, rows_v, col):
                pltpu.sync_copy(idx_hbm.at[pl.ds(toff, tail)],
                                idx_v.at[pl.ds(0, tail)])
                cps = [pltpu.async_copy(
                    h_hbm.at[idx_v.at[pl.ds(st, ln)]],
                    rows_v.at[pl.ds(st, ln)], sem_g)
                    for (st, ln) in tail_slices]
                for cp in cps:
                    cp.wait()
                pltpu.sync_copy(rows_v.at[pl.ds(0, tail)],
                                cat_hbm.at[pl.ds(toff - e0, tail),
                                           pl.ds(col, _EMB)])
            one(dst_hbm, idx_a, rows_a, 0)
            one(src_hbm, idx_b, rows_b, _EMB)

    return k(h64, src, dst)


def _scatter(wmsg2, dst, e0, n):
    rows_per_s = n // 2 // _NS
    n_full = rows_per_s // 128
    tail = rows_per_s - n_full * 128

    @functools.partial(
        pl.kernel,
        out_type=jax.ShapeDtypeStruct((_HALF, 2 * _EMB), jnp.float32),
        mesh=_sc_mesh(),
        scratch_types=[
            pltpu.VMEM((256,), jnp.int32),
            pltpu.VMEM((2, 128), jnp.int32),
            pltpu.VMEM((128, _EMB), jnp.float32),
            pltpu.VMEM((128, _EMB), jnp.float32),
            pltpu.VMEM_SHARED((_ACC_ROWS, _EMB), jnp.float32),
            pltpu.SemaphoreType.DMA,
            pltpu.SemaphoreType.DMA,
        ],
        compiler_params=pltpu.CompilerParams(use_tc_tiling_on_sc=False),
    )
    def k(w_hbm, d_hbm, out_hbm, raw_v, idx2_v, vals_v, vals2_v, acc,
          sem_r, sem_a):
        c = lax.axis_index("c")
        s = lax.axis_index("s")
        lane = lax.iota(jnp.int32, 16)
        row_base = s * rows_per_s
        nodes0 = c * _HALF

        def fixup(kv, nvalid):
            v = raw_v[pl.ds(kv * 16, 16)]
            local = v - nodes0
            inr = (local >= 0) & (local < _HALF)
            if nvalid < 16:
                inr = inr & (lane < nvalid)
            lrow = (local >> 1) + (local & 1) * _ODD_BASE
            pad = _PAD_BASE + s * 16 + ((lane + kv) & 15)
            idx2_v[kv // 8, pl.ds((kv % 8) * 16, 16)] = (
                jnp.where(inr, lrow, pad))

        def zrow(r, carry):
            for t in range(_EMB // 16):
                vals_v[r, pl.ds(t * 16, 16)] = jnp.zeros((16,), jnp.float32)
            return carry
        lax.fori_loop(0, 128, zrow, 0)
        zb = s * (_ACC_ROWS // _NS)
        for t in range(_ACC_ROWS // _NS // 128):
            pltpu.sync_copy(vals_v, acc.at[pl.ds(zb + t * 128, 128)])
        rem = (_ACC_ROWS // _NS) % 128
        if rem:
            pltpu.sync_copy(
                vals_v.at[pl.ds(0, rem)],
                acc.at[pl.ds(zb + (_ACC_ROWS // _NS) - rem, rem)])
        plsc.subcore_barrier()

        def chunk(roff, nrows):
            cv1 = pltpu.async_copy(
                w_hbm.at[pl.ds(roff, nrows), pl.ds(0, _EMB)],
                vals_v.at[pl.ds(0, nrows)], sem_r)
            cv2 = pltpu.async_copy(
                w_hbm.at[pl.ds(roff, nrows), pl.ds(_EMB, _EMB)],
                vals2_v.at[pl.ds(0, nrows)], sem_r)
            ci1 = pltpu.async_copy(d_hbm.at[pl.ds(e0 + roff, nrows)],
                                   raw_v.at[pl.ds(0, nrows)], sem_r)
            ci2 = pltpu.async_copy(
                d_hbm.at[pl.ds(e0 + n // 2 + roff, nrows)],
                raw_v.at[pl.ds(128, nrows)], sem_r)
            ci1.wait()
            ci2.wait()
            for half in range(2):
                for kv in range(8):
                    nvalid = max(0, min(16, nrows - kv * 16))
                    fixup(half * 8 + kv, nvalid)
            cv1.wait()
            cv2.wait()
            a1 = pltpu.async_copy(vals_v, acc.at[idx2_v.at[0]], sem_a,
                                  add=True)
            a2 = pltpu.async_copy(vals2_v, acc.at[idx2_v.at[1]], sem_a,
                                  add=True)
            a1.wait()
            a2.wait()

        def step(i, carry):
            chunk(row_base + i * 128, 128)
            return carry

        lax.fori_loop(0, n_full, step, 0)
        if tail:
            chunk(row_base + n_full * 128, tail)
        plsc.subcore_barrier()

        rows = _QUART // 4
        @pl.when(s < 4)
        def _():
            pltpu.sync_copy(
                acc.at[pl.ds(s * rows, rows)],
                out_hbm.at[pl.ds(c * _QUART + s * rows, rows),
                           pl.ds(0, _EMB)])

        @pl.when((s >= 4) & (s < 8))
        def _():
            pltpu.sync_copy(
                acc.at[pl.ds(_ODD_BASE + (s - 4) * rows, rows)],
                out_hbm.at[pl.ds(c * _QUART + (s - 4) * rows, rows),
                           pl.ds(_EMB, _EMB)])

    return k(wmsg2, dst)


def _full_spec(d1, d2):
    return pl.BlockSpec((d1, d2), lambda i: (0, 0))


def _row_spec(blk, d):
    return pl.BlockSpec((blk, d), lambda i: (i, 0))


def _lnorm(x, eps=1e-5):
    m = jnp.mean(x, axis=-1, keepdims=True)
    v = jnp.mean((x - m) ** 2, axis=-1, keepdims=True)
    return (x - m) * lax.rsqrt(v + eps)


def _bd(w):
    z = jnp.zeros_like(w)
    return jnp.concatenate(
        [jnp.concatenate([w, z], 1), jnp.concatenate([z, w], 1)], 0)


def _dup(b):
    return jnp.concatenate([b, b]).reshape(1, -1)


def _enc_body(oe, oo, w1, b1, g1, be1, w2, b2, out):
    def half(o):
        h = jnp.dot(o[...], w1[...], preferred_element_type=jnp.float32)
        h = _lnorm(h + b1[...]) * g1[...] + be1[...]
        h = jnp.maximum(h, 0.0)
        h2 = jnp.dot(h, w2[...], preferred_element_type=jnp.float32)
        return jnp.maximum(h2 + b2[...], 0.0)
    out[...] = jnp.concatenate([half(oe), half(oo)], axis=-1)


def _encoder(obs_e, obs_o, p):
    return pl.pallas_call(
        _enc_body,
        grid=(_HALF // _BLK_N,),
        in_specs=[
            _row_spec(_BLK_N, 10), _row_spec(_BLK_N, 10),
            _full_spec(10, _HID), _full_spec(1, _HID),
            _full_spec(1, _HID), _full_spec(1, _HID),
            _full_spec(_HID, _EMB), _full_spec(1, _EMB),
        ],
        out_specs=_row_spec(_BLK_N, 2 * _EMB),
        out_shape=jax.ShapeDtypeStruct((_HALF, 2 * _EMB), jnp.float32),
    )(obs_e, obs_o, p['enc_w1'], p['enc_b1'].reshape(1, -1),
      p['enc_g1'].reshape(1, -1), p['enc_be1'].reshape(1, -1),
      p['enc_w2'], p['enc_b2'].reshape(1, -1))


def _edge_body(cat1, cat2, eaT1, eaT2, wij, we, b1, w2, b2, out):
    def part(cat, eaT):
        pre = jnp.dot(cat[...], wij[...], preferred_element_type=jnp.float32)
        pre = pre + lax.dot_general(
            eaT[...], we[...], (((0,), (0,)), ((), ())),
            preferred_element_type=jnp.float32)
        h1 = jnp.maximum(pre + b1[...], 0.0)
        msg = jnp.dot(h1, w2[...], preferred_element_type=jnp.float32)
        return 0.25 * (msg + b2[...])
    out[...] = jnp.concatenate([part(cat1, eaT1), part(cat2, eaT2)], -1)


def _edge_mlp(cat, eaT, e0, n, wij, we, b1, w2, b2):
    hb = n // 2 // _BLK_E
    ea1 = e0 // _BLK_E
    ea2 = (e0 + n // 2) // _BLK_E
    return pl.pallas_call(
        _edge_body,
        grid=(hb,),
        in_specs=[
            pl.BlockSpec((_BLK_E, 2 * _EMB), lambda i: (i, 0)),
            pl.BlockSpec((_BLK_E, 2 * _EMB), lambda i, _hb=hb: (i + _hb, 0)),
            pl.BlockSpec((_EDGE, _BLK_E), lambda i, _o=ea1: (0, i + _o)),
            pl.BlockSpec((_EDGE, _BLK_E), lambda i, _o=ea2: (0, i + _o)),
            _full_spec(2 * _EMB, _HID), _full_spec(_EDGE, _HID),
            _full_spec(1, _HID),
            _full_spec(_HID, _EMB), _full_spec(1, _EMB),
        ],
        out_specs=_row_spec(_BLK_E, 2 * _EMB),
        out_shape=jax.ShapeDtypeStruct((n // 2, 2 * _EMB), jnp.float32),
    )(cat, cat, eaT, eaT, wij, we, b1.reshape(1, -1), w2, b2.reshape(1, -1))


def _upd_body(h, aa, ab, w1h, w1a, b1, w2, b2, g, b, out):
    u = jnp.dot(h[...], w1h[...], preferred_element_type=jnp.float32)
    u = u + jnp.dot(aa[...] + ab[...], w1a[...],
                    preferred_element_type=jnp.float32)
    u = jnp.maximum(u + b1[...], 0.0)
    upd = jnp.dot(u, w2[...], preferred_element_type=jnp.float32) + b2[...]
    y = h[...] + upd
    yl = jnp.concatenate(
        [_lnorm(y[:, :_EMB]), _lnorm(y[:, _EMB:])], axis=-1)
    out[...] = yl * g[...] + b[...]


def _update(h_p, aggr_a, aggr_b, bw1h, bw1a, b1, bw2, b2, g, b):
    return pl.pallas_call(
        _upd_body,
        grid=(_HALF // _BLK_N,),
        in_specs=[
            _row_spec(_BLK_N, 2 * _EMB), _row_spec(_BLK_N, 2 * _EMB),
            _row_spec(_BLK_N, 2 * _EMB),
            _full_spec(2 * _EMB, 2 * _HID), _full_spec(2 * _EMB, 2 * _HID),
            _full_spec(1, 2 * _HID),
            _full_spec(2 * _HID, 2 * _EMB), _full_spec(1, 2 * _EMB),
            _full_spec(1, 2 * _EMB), _full_spec(1, 2 * _EMB),
        ],
        out_specs=_row_spec(_BLK_N, 2 * _EMB),
        out_shape=jax.ShapeDtypeStruct((_HALF, 2 * _EMB), jnp.float32),
    )(h_p, aggr_a, aggr_b, bw1h, bw1a, b1, bw2, b2, g, b)


def _out_body(h, w1, b1, w2, b2, out):
    u = jnp.dot(h[...], w1[...], preferred_element_type=jnp.float32)
    u = jnp.maximum(u + b1[...], 0.0)
    out[...] = jnp.dot(u, w2[...], preferred_element_type=jnp.float32) + b2[...]


def _output(h_p, bw1, b1, bw2, b2):
    return pl.pallas_call(
        _out_body,
        grid=(_HALF // _BLK_N,),
        in_specs=[
            _row_spec(_BLK_N, 2 * _EMB),
            _full_spec(2 * _EMB, 2 * _HID), _full_spec(1, 2 * _HID),
            _full_spec(2 * _HID, 2 * _EMB), _full_spec(1, 2 * _EMB),
        ],
        out_specs=_row_spec(_BLK_N, 2 * _EMB),
        out_shape=jax.ShapeDtypeStruct((_HALF, 2 * _EMB), jnp.float32),
    )(h_p, bw1, b1, bw2, b2)


def kernel(obs, edge_index, edge_attr, params):
    p = params
    src = edge_index[0]
    dst = edge_index[1]
    obs_e = obs[0::2, :10]
    obs_o = obs[1::2, :10]
    eaT = edge_attr.T

    h_p = _encoder(obs_e, obs_o, p)
    for lp in p['layers']:
        h64 = h_p.reshape(_N, _EMB)
        w1 = lp['msg_w1']
        cat_a = _gather(h64, src, dst, 0, _EA)
        cat_b = _gather(h64, src, dst, _EA, _EB)
        wmsg_a = _edge_mlp(cat_a, eaT, 0, _EA, w1[:2 * _EMB], w1[2 * _EMB:],
                           lp['msg_b1'], lp['msg_w2'], lp['msg_b2'])
        aggr_a = _scatter(wmsg_a, dst, 0, _EA)
        wmsg_b = _edge_mlp(cat_b, eaT, _EA, _EB, w1[:2 * _EMB],
                           w1[2 * _EMB:], lp['msg_b1'], lp['msg_w2'],
                           lp['msg_b2'])
        aggr_b = _scatter(wmsg_b, dst, _EA, _EB)
        uw1 = lp['upd_w1']
        h_p = _update(h_p, aggr_a, aggr_b,
                      _bd(uw1[:_EMB]), _bd(uw1[_EMB:]),
                      _dup(lp['upd_b1']), _bd(lp['upd_w2']),
                      _dup(lp['upd_b2']), _dup(lp['ln_g']),
                      _dup(lp['ln_b']))
    out_p = _output(h_p, _bd(p['out_w1']), _dup(p['out_b1']),
                    _bd(p['out_w2']), _dup(p['out_b2']))
    return out_p.reshape(_N, _EMB)
